# Initial kernel scaffold; baseline (speedup 1.0000x reference)
import jax
import jax.numpy as jnp
from jax.experimental import pallas as pl


def kernel(x, edge_index, W):
    raise NotImplementedError("write your pallas kernel here")


# speedup vs baseline: 51.0946x; 51.0946x over previous
import functools

import jax
import jax.numpy as jnp
from jax import lax
from jax.experimental import pallas as pl
from jax.experimental.pallas import tpu as pltpu
from jax.experimental.pallas import tpu_sc as plsc

N = 10000
E = 320000
D = 128
H = 4
DH = 32
INV_SQRT_DH = 0.17677669529663687

NC, NS = 2, 16
NW = NC * NS
EPW = E // NW
C = 80
NCHUNK = EPW // C
G = C // 16

_MESH = plsc.VectorSubcoreMesh(
    core_axis_name="c", subcore_axis_name="s", num_cores=NC, num_subcores=NS
)


def _mm_body(x_ref, w_ref, o_ref):
    o_ref[...] = jnp.dot(x_ref[...], w_ref[...],
                         preferred_element_type=jnp.float32)


def _project(x, W):
    return pl.pallas_call(
        _mm_body,
        grid=(10,),
        in_specs=[
            pl.BlockSpec((N // 10, D), lambda i: (i, 0)),
            pl.BlockSpec((D, D), lambda i: (0, 0)),
        ],
        out_specs=pl.BlockSpec((N // 10, D), lambda i: (i, 0)),
        out_shape=jax.ShapeDtypeStruct((N, D), jnp.float32),
    )(x, W)


def _add_body(a_ref, b_ref, o_ref):
    o_ref[...] = a_ref[...] + b_ref[...]


def _combine(p0, p1):
    return pl.pallas_call(
        _add_body,
        grid=(10,),
        in_specs=[
            pl.BlockSpec((N // 10, D), lambda i: (i, 0)),
            pl.BlockSpec((N // 10, D), lambda i: (i, 0)),
        ],
        out_specs=pl.BlockSpec((N // 10, D), lambda i: (i, 0)),
        out_shape=jax.ShapeDtypeStruct((N, D), jnp.float32),
    )(p0, p1)


def _norm_body(p0_ref, p1_ref, s0_ref, s1_ref, o_ref):
    s = jnp.maximum(s0_ref[...] + s1_ref[...], jnp.float32(1e-30))
    inv = (jnp.float32(1.0) / s).reshape(N // 10, H, 1)
    p = (p0_ref[...] + p1_ref[...]).reshape(N // 10, H, DH)
    o_ref[...] = (p * inv).reshape(N // 10, D)


def _norm_combine(p0, p1, s0, s1):
    return pl.pallas_call(
        _norm_body,
        grid=(10,),
        in_specs=[
            pl.BlockSpec((N // 10, D), lambda i: (i, 0)),
            pl.BlockSpec((N // 10, D), lambda i: (i, 0)),
            pl.BlockSpec((N // 10, H), lambda i: (i, 0)),
            pl.BlockSpec((N // 10, H), lambda i: (i, 0)),
        ],
        out_specs=pl.BlockSpec((N // 10, D), lambda i: (i, 0)),
        out_shape=jax.ShapeDtypeStruct((N, D), jnp.float32),
    )(p0, p1, s0.reshape(N, H), s1.reshape(N, H))


def _zero_vec(ref, nelem):
    z = jnp.zeros((16,), jnp.float32)

    def body(i, _):
        ref[pl.ds(i * 16, 16)] = z
        return 0

    lax.fori_loop(0, nelem // 16, body, 0)


@functools.partial(
    pl.kernel,
    out_type=(
        jax.ShapeDtypeStruct((E * H,), jnp.float32),
        jax.ShapeDtypeStruct((NC * N * H,), jnp.float32),
    ),
    mesh=_MESH,
    compiler_params=pltpu.CompilerParams(needs_layout_passes=False),
    scratch_types=[
        pltpu.VMEM((EPW,), jnp.int32),
        pltpu.VMEM((EPW,), jnp.int32),
        pltpu.VMEM((C, D), jnp.float32),
        pltpu.VMEM((C, D), jnp.float32),
        pltpu.VMEM((C, D), jnp.float32),
        pltpu.VMEM((C, D), jnp.float32),
        pltpu.VMEM((H * 16 * (C + 1),), jnp.float32),
        pltpu.VMEM((EPW * H,), jnp.float32),
        pltpu.VMEM((H * C,), jnp.float32),
        pltpu.VMEM((H * C,), jnp.int32),
        pltpu.VMEM((2560,), jnp.float32),
        pltpu.VMEM_SHARED((N * H,), jnp.float32),
        pltpu.SemaphoreType.DMA,
        pltpu.SemaphoreType.DMA,
    ],
)
def _attn_scores(ft_hbm, src_hbm, dst_hbm, ee_hbm, sp_hbm,
                 src_ids, dst_ids, srows_a, drows_a, srows_b, drows_b,
                 pt, ee_all, ee_chunk, sidx, zbuf, s_acc, sem_a, sem_b):
    cid = lax.axis_index("c")
    sid = lax.axis_index("s")
    wid = cid * NS + sid
    e0 = wid * EPW

    pltpu.sync_copy(src_hbm.at[pl.ds(e0, EPW)], src_ids)
    pltpu.sync_copy(dst_hbm.at[pl.ds(e0, EPW)], dst_ids)

    _zero_vec(zbuf, 2560)

    @pl.when(sid < NS - 1)
    def _():
        pltpu.sync_copy(zbuf, s_acc.at[pl.ds(sid * 2560, 2560)])

    @pl.when(sid == NS - 1)
    def _():
        pltpu.sync_copy(zbuf.at[pl.ds(0, 1600)],
                        s_acc.at[pl.ds(38400, 1600)])

    plsc.subcore_barrier()

    lanes = lax.iota(jnp.int32, 16)
    ST = C + 1
    lst = lanes * ST

    def start_gathers(j, sr, dr, sem):
        cb = j * C
        pltpu.async_copy(ft_hbm.at[src_ids.at[pl.ds(cb, C)]], sr, sem)
        pltpu.async_copy(ft_hbm.at[dst_ids.at[pl.ds(cb, C)]], dr, sem)

    def wait_gathers(sr, dr, sem):
        pltpu.make_async_copy(ft_hbm.at[src_ids.at[pl.ds(0, C)]], sr,
                              sem).wait()
        pltpu.make_async_copy(ft_hbm.at[dst_ids.at[pl.ds(0, C)]], dr,
                              sem).wait()

    def compute(j, sr, dr):
        cb = j * C

        def prod_body(e, _):
            for h in range(H):
                v = (sr[e, pl.ds(2 * h * 16, 16)]
                     * dr[e, pl.ds(2 * h * 16, 16)]
                     + sr[e, pl.ds((2 * h + 1) * 16, 16)]
                     * dr[e, pl.ds((2 * h + 1) * 16, 16)])
                plsc.store_scatter(pt, [lst + (h * 16 * ST + e)], v)
            return 0

        lax.fori_loop(0, C, prod_body, 0)

        def group_body(g, _):
            d16 = dst_ids[pl.ds(cb + g * 16, 16)]
            for h in range(H):
                acc = pt[pl.ds(h * 16 * ST + g * 16, 16)]
                for t in range(1, 16):
                    acc = acc + pt[pl.ds((h * 16 + t) * ST + g * 16, 16)]
                eeh = jnp.exp(acc * INV_SQRT_DH)
                ee_all[pl.ds((j * H + h) * C + g * 16, 16)] = eeh
                ee_chunk[pl.ds(h * C + g * 16, 16)] = eeh
                sidx[pl.ds(h * C + g * 16, 16)] = d16 * H + h
            return 0

        lax.fori_loop(0, G, group_body, 0)

        pltpu.sync_copy(ee_chunk, s_acc.at[sidx], add=True)

    start_gathers(0, srows_a, drows_a, sem_a)

    def pair_body(jj, _):
        j0 = jj * 2
        start_gathers(j0 + 1, srows_b, drows_b, sem_b)
        wait_gathers(srows_a, drows_a, sem_a)
        compute(j0, srows_a, drows_a)
        start_gathers(j0 + 2, srows_a, drows_a, sem_a)
        wait_gathers(srows_b, drows_b, sem_b)
        compute(j0 + 1, srows_b, drows_b)
        return 0

    lax.fori_loop(0, (NCHUNK - 1) // 2, pair_body, 0)
    wait_gathers(srows_a, drows_a, sem_a)
    compute(NCHUNK - 1, srows_a, drows_a)

    pltpu.sync_copy(ee_all, ee_hbm.at[pl.ds(wid * EPW * H, EPW * H)])
    plsc.subcore_barrier()

    @pl.when(sid == 0)
    def _():
        pltpu.sync_copy(s_acc, ee_all)
        pltpu.sync_copy(ee_all, sp_hbm.at[pl.ds(cid * N * H, N * H)])


@functools.partial(
    pl.kernel,
    out_type=jax.ShapeDtypeStruct((NC, N, D), jnp.float32),
    mesh=_MESH,
    compiler_params=pltpu.CompilerParams(needs_layout_passes=False),
    scratch_types=[
        pltpu.VMEM((EPW,), jnp.int32),
        pltpu.VMEM((EPW,), jnp.int32),
        pltpu.VMEM((C, D), jnp.float32),
        pltpu.VMEM((C, D), jnp.float32),
        pltpu.VMEM((H * C + 16,), jnp.float32),
        pltpu.VMEM((H * C + 16,), jnp.float32),
        pltpu.VMEM((C,), jnp.int32),
        pltpu.VMEM_SHARED((N, D), jnp.float32),
        pltpu.SemaphoreType.DMA,
        pltpu.SemaphoreType.DMA,
    ],
)
def _aggregate(ft_hbm, src_hbm, dst_hbm, ee_hbm, pout_hbm,
               src_ids, dst_ids, srows_a, srows_b, ee_a, ee_b,
               didx, out_acc, sem_a, sem_b):
    cid = lax.axis_index("c")
    sid = lax.axis_index("s")
    wid = cid * NS + sid
    e0 = wid * EPW

    pltpu.sync_copy(src_hbm.at[pl.ds(e0, EPW)], src_ids)
    pltpu.sync_copy(dst_hbm.at[pl.ds(e0, EPW)], dst_ids)

    z = jnp.zeros((16,), jnp.float32)

    def zrow_body(i, _):
        for k in range(8):
            srows_a[i, pl.ds(k * 16, 16)] = z
        return 0

    lax.fori_loop(0, C, zrow_body, 0)
    r0 = sid * 624

    @pl.when(sid < NS - 1)
    def _():
        for k in range(7):
            pltpu.sync_copy(srows_a, out_acc.at[pl.ds(r0 + k * C, C), :])
        pltpu.sync_copy(srows_a.at[pl.ds(0, 64), :],
                        out_acc.at[pl.ds(r0 + 560, 64), :])

    @pl.when(sid == NS - 1)
    def _():
        for k in range(8):
            pltpu.sync_copy(srows_a, out_acc.at[pl.ds(9360 + k * C, C), :])

    plsc.subcore_barrier()

    def start_fetch(j, sr, eec, sem):
        cb = j * C
        pltpu.async_copy(ft_hbm.at[src_ids.at[pl.ds(cb, C)]], sr, sem)
        pltpu.async_copy(
            ee_hbm.at[pl.ds((wid * EPW + cb) * H, H * C)],
            eec.at[pl.ds(0, H * C)], sem)

    def wait_fetch(j, sr, eec, sem):
        cb = j * C
        pltpu.make_async_copy(ft_hbm.at[src_ids.at[pl.ds(cb, C)]], sr,
                              sem).wait()
        pltpu.make_async_copy(
            ee_hbm.at[pl.ds((wid * EPW + cb) * H, H * C)],
            eec.at[pl.ds(0, H * C)], sem).wait()

    def compute(j, sr, eec):
        cb = j * C

        def didx_body(g, _):
            didx[pl.ds(g * 16, 16)] = dst_ids[pl.ds(cb + g * 16, 16)]
            return 0

        lax.fori_loop(0, G, didx_body, 0)

        def scale_body(e, _):
            for h in range(H):
                av = jnp.broadcast_to(eec[pl.ds(h * C + e, 16)][0], (16,))
                for k in range(2 * h, 2 * h + 2):
                    sr[e, pl.ds(k * 16, 16)] = sr[e, pl.ds(k * 16, 16)] * av
            return 0

        lax.fori_loop(0, C, scale_body, 0)
        pltpu.sync_copy(sr, out_acc.at[didx], add=True)

    start_fetch(0, srows_a, ee_a, sem_a)

    def pair_body(jj, _):
        j0 = jj * 2
        start_fetch(j0 + 1, srows_b, ee_b, sem_b)
        wait_fetch(j0, srows_a, ee_a, sem_a)
        compute(j0, srows_a, ee_a)
        start_fetch(j0 + 2, srows_a, ee_a, sem_a)
        wait_fetch(j0 + 1, srows_b, ee_b, sem_b)
        compute(j0 + 1, srows_b, ee_b)
        return 0

    lax.fori_loop(0, (NCHUNK - 1) // 2, pair_body, 0)
    wait_fetch(NCHUNK - 1, srows_a, ee_a, sem_a)
    compute(NCHUNK - 1, srows_a, ee_a)
    plsc.subcore_barrier()

    @pl.when(sid < NS - 1)
    def _():
        pltpu.sync_copy(out_acc.at[pl.ds(r0, 624), :],
                        pout_hbm.at[cid, pl.ds(r0, 624), :])

    @pl.when(sid == NS - 1)
    def _():
        pltpu.sync_copy(out_acc.at[pl.ds(9360, 640), :],
                        pout_hbm.at[cid, pl.ds(9360, 640), :])


def kernel(x, edge_index, W):
    src = edge_index[0]
    dst = edge_index[1]
    ft = _project(x, W)
    ee, sparts = _attn_scores(ft, src, dst)
    pout = _aggregate(ft, src, dst, ee)
    return _norm_combine(pout[0], pout[1], sparts[:N * H], sparts[N * H:])
